# baseline (speedup 1.0000x reference)
import jax
import jax.numpy as jnp
from jax import lax
from jax.experimental import pallas as pl


_ROWS_PER_BLOCK = 512
_D = 1024


def _argmin_rows_kernel(x_ref, o_ref):
    xb = x_ref[...]
    minval = jnp.min(xb, axis=1, keepdims=True)
    iota = lax.broadcasted_iota(jnp.int32, xb.shape, 1)
    idx = jnp.min(jnp.where(xb == minval, iota, _D), axis=1)
    o_ref[...] = idx.astype(jnp.int32)


def kernel(x):
    b, d1, d2 = x.shape
    rows = b * d1
    xf = x.reshape(rows, d2)
    grid = rows // _ROWS_PER_BLOCK
    out = pl.pallas_call(
        _argmin_rows_kernel,
        grid=(grid,),
        in_specs=[pl.BlockSpec((_ROWS_PER_BLOCK, d2), lambda i: (i, 0))],
        out_specs=pl.BlockSpec((_ROWS_PER_BLOCK,), lambda i: (i,)),
        out_shape=jax.ShapeDtypeStruct((rows,), jnp.int32),
    )(xf)
    return out.reshape(b, d1)


# speedup vs baseline: 1.0443x; 1.0443x over previous
import jax
import jax.numpy as jnp
from jax import lax
from jax.experimental import pallas as pl


_ROWS_PER_BLOCK = 512
_D = 1024


def _argmin_rows_kernel(x_ref, o_ref):
    xb = x_ref[...]
    minval = jnp.min(xb, axis=1, keepdims=True)
    iota = lax.broadcasted_iota(jnp.int32, xb.shape, 1).astype(jnp.float32)
    idx = jnp.min(jnp.where(xb == minval, iota, float(_D)), axis=1)
    o_ref[...] = idx.astype(jnp.int32)


def kernel(x):
    b, d1, d2 = x.shape
    rows = b * d1
    xf = x.reshape(rows, d2)
    grid = rows // _ROWS_PER_BLOCK
    out = pl.pallas_call(
        _argmin_rows_kernel,
        grid=(grid,),
        in_specs=[pl.BlockSpec((_ROWS_PER_BLOCK, d2), lambda i: (i, 0))],
        out_specs=pl.BlockSpec((_ROWS_PER_BLOCK,), lambda i: (i,)),
        out_shape=jax.ShapeDtypeStruct((rows,), jnp.int32),
    )(xf)
    return out.reshape(b, d1)


# speedup vs baseline: 1.5980x; 1.5303x over previous
import jax
import jax.numpy as jnp
from jax import lax
from jax.experimental import pallas as pl


_ROWS_PER_BLOCK = 2048
_D = 1024


def _argmin_rows_kernel(x_ref, o_ref):
    xb = x_ref[...]
    minval = jnp.min(xb, axis=1, keepdims=True)
    iota = lax.broadcasted_iota(jnp.int32, xb.shape, 1).astype(jnp.float32)
    idx = jnp.min(jnp.where(xb == minval, iota, float(_D)), axis=1)
    o_ref[...] = idx.astype(jnp.int32)


def kernel(x):
    b, d1, d2 = x.shape
    rows = b * d1
    xf = x.reshape(rows, d2)
    grid = rows // _ROWS_PER_BLOCK
    out = pl.pallas_call(
        _argmin_rows_kernel,
        grid=(grid,),
        in_specs=[pl.BlockSpec((_ROWS_PER_BLOCK, d2), lambda i: (i, 0))],
        out_specs=pl.BlockSpec((_ROWS_PER_BLOCK,), lambda i: (i,)),
        out_shape=jax.ShapeDtypeStruct((rows,), jnp.int32),
    )(xf)
    return out.reshape(b, d1)


# speedup vs baseline: 1.6901x; 1.0576x over previous
import jax
import jax.numpy as jnp
from jax import lax
from jax.experimental import pallas as pl


_ROWS_PER_BLOCK = 4096
_D = 1024


def _argmin_rows_kernel(x_ref, o_ref):
    xb = x_ref[...]
    minval = jnp.min(xb, axis=1, keepdims=True)
    iota = lax.broadcasted_iota(jnp.int32, xb.shape, 1).astype(jnp.float32)
    idx = jnp.min(jnp.where(xb == minval, iota, float(_D)), axis=1)
    o_ref[...] = idx.astype(jnp.int32)


def kernel(x):
    b, d1, d2 = x.shape
    rows = b * d1
    xf = x.reshape(rows, d2)
    grid = rows // _ROWS_PER_BLOCK
    out = pl.pallas_call(
        _argmin_rows_kernel,
        grid=(grid,),
        in_specs=[pl.BlockSpec((_ROWS_PER_BLOCK, d2), lambda i: (i, 0))],
        out_specs=pl.BlockSpec((_ROWS_PER_BLOCK,), lambda i: (i,)),
        out_shape=jax.ShapeDtypeStruct((rows,), jnp.int32),
    )(xf)
    return out.reshape(b, d1)
